# baseline (speedup 1.0000x reference)
import functools

import numpy as np

import jax
import jax.numpy as jnp
from jax import lax
from jax.experimental import pallas as pl
from jax.experimental.pallas import tpu as pltpu
from jax.experimental.pallas import tpu_sc as plsc

DIM = 64
NROW = 1000000
BATCH = 16384
NC, NS, L = 2, 16, 16
NW = NC * NS
PER_W = BATCH // NW
CHUNK = 128
N_CHUNKS = PER_W // CHUNK
GROUPS = CHUNK // L

TBLK = 512
OBLK = TBLK // 2
TGRID = -(-NROW // TBLK)


def _transpose_body(pe_ref, po_ref, in_ref, out_ref):
    x = in_ref[...]
    valid = NROW - pl.program_id(0) * TBLK
    col = lax.broadcasted_iota(jnp.int32, (DIM, TBLK), 1)
    x = jnp.where(col < valid, x, 0.0)
    dn = (((1,), (1,)), ((), ()))
    left = lax.dot_general(pe_ref[...], x, dn,
                           precision=lax.Precision.HIGHEST)
    right = lax.dot_general(po_ref[...], x, dn,
                            precision=lax.Precision.HIGHEST)
    out_ref[:, 0:DIM] = left
    out_ref[:, DIM:2 * DIM] = right


def _make_row_major(table_t, pe, po):
    return pl.pallas_call(
        _transpose_body,
        grid=(TGRID,),
        in_specs=[
            pl.BlockSpec((OBLK, TBLK), lambda i: (0, 0)),
            pl.BlockSpec((OBLK, TBLK), lambda i: (0, 0)),
            pl.BlockSpec((DIM, TBLK), lambda i: (0, i)),
        ],
        out_specs=pl.BlockSpec((OBLK, 2 * DIM), lambda i: (i, 0)),
        out_shape=jax.ShapeDtypeStruct((NROW // 2, 2 * DIM), jnp.float32),
    )(pe, po, table_t)


def _transe_body(post_hbm, negt_hbm, ent_hbm, rel_hbm, pos_out, neg_out,
                 ihv, irv, itv, hb, rb, tb, part, outb, sem):
    wid = lax.axis_index("s") * NC + lax.axis_index("c")
    base = wid * PER_W

    def one_batch(tript_hbm, out_hbm):
        @pl.loop(0, N_CHUNKS)
        def _chunk(c):
            start = base + c * CHUNK
            pltpu.sync_copy(tript_hbm.at[0, pl.ds(start, CHUNK)], ihv)
            pltpu.sync_copy(tript_hbm.at[1, pl.ds(start, CHUNK)], irv)
            pltpu.sync_copy(tript_hbm.at[2, pl.ds(start, CHUNK)], itv)
            ch = pltpu.async_copy(ent_hbm.at[ihv], hb, sem)
            cr = pltpu.async_copy(rel_hbm.at[irv], rb, sem)
            ct = pltpu.async_copy(ent_hbm.at[itv], tb, sem)
            ch.wait()
            cr.wait()
            ct.wait()

            @pl.loop(0, GROUPS)
            def _group(g):
                row0 = g * L
                for t in range(L):
                    row = row0 + t
                    acc = None
                    for d in range(DIM // L):
                        sl = pl.ds(d * L, L)
                        v = jnp.abs(hb[row, sl] + rb[row, sl] - tb[row, sl])
                        acc = v if acc is None else acc + v
                    part[pl.ds(t * L, L)] = acc
                rows = lax.iota(jnp.int32, L) * L
                red = plsc.load_gather(part, [rows])
                for j in range(1, L):
                    red = red + plsc.load_gather(part, [rows + j])
                outb[pl.ds(row0, L)] = red

            pltpu.sync_copy(outb, out_hbm.at[pl.ds(start, CHUNK)])

    one_batch(post_hbm, pos_out)
    one_batch(negt_hbm, neg_out)


def _selection_mats():
    pe = np.zeros((OBLK, TBLK), np.float32)
    po = np.zeros((OBLK, TBLK), np.float32)
    pe[np.arange(OBLK), 2 * np.arange(OBLK)] = 1.0
    po[np.arange(OBLK), 2 * np.arange(OBLK) + 1] = 1.0
    return jnp.asarray(pe), jnp.asarray(po)


@jax.jit
def kernel(positive_triplets, negative_triplets, entities_emb, relations_emb):
    pos_t = positive_triplets.astype(jnp.int32).T
    neg_t = negative_triplets.astype(jnp.int32).T
    pe, po = _selection_mats()
    ent_rm = _make_row_major(entities_emb.T, pe, po).reshape(NROW, DIM)
    rel_rm = _make_row_major(relations_emb.T, pe, po).reshape(NROW, DIM)

    mesh = plsc.VectorSubcoreMesh(
        core_axis_name="c", subcore_axis_name="s",
        num_cores=NC, num_subcores=NS)
    run = pl.kernel(
        _transe_body,
        out_type=(jax.ShapeDtypeStruct((BATCH,), jnp.float32),
                  jax.ShapeDtypeStruct((BATCH,), jnp.float32)),
        mesh=mesh,
        compiler_params=pltpu.CompilerParams(
            needs_layout_passes=False, use_tc_tiling_on_sc=False),
        scratch_types=[
            pltpu.VMEM((CHUNK,), jnp.int32),
            pltpu.VMEM((CHUNK,), jnp.int32),
            pltpu.VMEM((CHUNK,), jnp.int32),
            pltpu.VMEM((CHUNK, DIM), jnp.float32),
            pltpu.VMEM((CHUNK, DIM), jnp.float32),
            pltpu.VMEM((CHUNK, DIM), jnp.float32),
            pltpu.VMEM((L * L,), jnp.float32),
            pltpu.VMEM((CHUNK,), jnp.float32),
            pltpu.SemaphoreType.DMA,
        ],
    )
    return run(pos_t, neg_t, ent_rm, rel_rm)
